# Initial kernel scaffold; baseline (speedup 1.0000x reference)
import jax
import jax.numpy as jnp
from jax.experimental import pallas as pl


def kernel(x_user, x_server, edge_index_u2s, edge_index_s2u, edge_attr_u2s, edge_attr_s2u, params):
    raise NotImplementedError("write your pallas kernel here")


# speedup vs baseline: 8.5370x; 8.5370x over previous
import dataclasses
import functools

import jax
import jax.numpy as jnp
from jax import lax
from jax.experimental import pallas as pl
from jax.experimental.pallas import tpu as pltpu
from jax.experimental.pallas import tpu_sc as plsc

f32 = jnp.float32
i32 = jnp.int32

H = 128
ALPHA = 0.2
N = 10000
E = 160000
BN = 2000
BE = 2000
NBE = E // BE
NSEGP = 10240
EC = E // 16
GP = EC // 16
COLS = NSEGP // 16

_MESH1 = dict(core_axis_name="c", subcore_axis_name="s", num_cores=1)
_MESH2 = dict(core_axis_name="c", subcore_axis_name="s")


def _sc_params():
    cp = pltpu.CompilerParams()
    if "needs_layout_passes" in pltpu.CompilerParams.__dataclass_fields__:
        cp = dataclasses.replace(cp, needs_layout_passes=False)
    return cp


def _relu(x):
    return jnp.maximum(x, 0.0)


def _leaky(x):
    return jnp.maximum(x, ALPHA * x)


def _dot(a, b):
    return jnp.dot(a, b, preferred_element_type=f32)


def _rows(b, d):
    return pl.BlockSpec((b, d), lambda i: (i, 0))


def _bcast(shape):
    return pl.BlockSpec(shape, lambda i: tuple(0 for _ in shape))


def _sds(shape):
    return jax.ShapeDtypeStruct(shape, f32)


def _tc_mlp2(x, w1, b1, w2, b2):
    def body(x_ref, w1_ref, b1_ref, w2_ref, b2_ref, o_ref):
        h = _relu(_dot(x_ref[...], w1_ref[...]) + b1_ref[...])
        o_ref[...] = _relu(_dot(h, w2_ref[...]) + b2_ref[...])

    return pl.pallas_call(
        body, grid=(N // BN,),
        in_specs=[_rows(BN, H), _bcast((H, H)), _bcast((1, H)),
                  _bcast((H, H)), _bcast((1, H))],
        out_specs=_rows(BN, H),
        out_shape=_sds((N, H)),
    )(x, w1, b1, w2, b2)


def _tc_node_pre(x, w, b, out_dims):
    D = sum(out_dims)

    def body(x_ref, w_ref, b_ref, *outs):
        y = _dot(x_ref[...], w_ref[...]) + b_ref[...]
        off = 0
        for o_ref, d in zip(outs, out_dims):
            o_ref[...] = y[:, off:off + d]
            off += d

    return pl.pallas_call(
        body, grid=(N // BN,),
        in_specs=[_rows(BN, H), _bcast((H, D)), _bcast((1, D))],
        out_specs=[_rows(BN, d) for d in out_dims],
        out_shape=[_sds((N, d)) for d in out_dims],
    )(x, w, b)


def _tc_att_u2s(qp, rg, a1w, a1b):
    def body(qp_ref, rg_ref, a1w_ref, a1b_ref, lg_ref):
        ah = _relu(qp_ref[:, :H] + rg_ref[:, :H])
        lg_ref[...] = _leaky(_dot(ah, a1w_ref[...]) + a1b_ref[...])

    return pl.pallas_call(
        body, grid=(NBE,),
        in_specs=[_rows(BE, qp.shape[1]), _rows(BE, rg.shape[1]),
                  _bcast((H, 1)), _bcast((1, 1))],
        out_specs=_rows(BE, 1),
        out_shape=_sds((E, 1)),
    )(qp, rg, a1w, a1b)


def _tc_out_u2s(qp, attr, w, m0r):
    def body(qp_ref, a_ref, w_ref, m_ref, o_ref):
        o_ref[...] = w_ref[...] * _relu(qp_ref[:, H:] + a_ref[...] * m_ref[...])

    return pl.pallas_call(
        body, grid=(NBE,),
        in_specs=[_rows(BE, 2 * H), _rows(BE, 1), _rows(BE, 1), _bcast((1, H))],
        out_specs=_rows(BE, H),
        out_shape=_sds((E, H)),
    )(qp, attr, w, m0r)


def _tc_msg_s2u_l0(pm, attr, w, m1r, m2w, m2b):
    def body(pm_ref, a_ref, w_ref, m1_ref, m2w_ref, m2b_ref, o_ref):
        h1 = _relu(pm_ref[...] + a_ref[...] * m1_ref[...])
        o_ref[...] = w_ref[...] * _relu(_dot(h1, m2w_ref[...]) + m2b_ref[...])

    return pl.pallas_call(
        body, grid=(NBE,),
        in_specs=[_rows(BE, H), _rows(BE, 1), _rows(BE, 1), _bcast((1, H)),
                  _bcast((H, H)), _bcast((1, H))],
        out_specs=_rows(BE, H),
        out_shape=_sds((E, H)),
    )(pm, attr, w, m1r, m2w, m2b)


def _tc_msg_s2u_l1(pm, prel, attr, w, m1r, m2w, m2b, r1m, r1c, r2w, r2b,
                   c0w, c0b, c1w, c1b):
    def body(pm_ref, pr_ref, a_ref, w_ref, m1_ref, m2w_ref, m2b_ref,
             r1m_ref, r1c_ref, r2w_ref, r2b_ref, c0w_ref, c0b_ref,
             c1w_ref, c1b_ref, o_ref, cl_ref):
        h1 = _relu(pm_ref[...] + a_ref[...] * m1_ref[...])
        out = w_ref[...] * _relu(_dot(h1, m2w_ref[...]) + m2b_ref[...])
        o_ref[...] = out
        r1 = _relu(pr_ref[...] + _dot(out, r1m_ref[...])
                   + a_ref[...] * r1c_ref[...])
        sattr = _relu(_dot(r1, r2w_ref[...]) + r2b_ref[...])
        c1 = _relu(_dot(sattr, c0w_ref[...]) + c0b_ref[...])
        cl_ref[...] = _leaky(_dot(c1, c1w_ref[...]) + c1b_ref[...])

    return pl.pallas_call(
        body, grid=(NBE,),
        in_specs=[_rows(BE, H), _rows(BE, H), _rows(BE, 1), _rows(BE, 1),
                  _bcast((1, H)), _bcast((H, H)), _bcast((1, H)),
                  _bcast((H, H)), _bcast((1, H)), _bcast((H, H)), _bcast((1, H)),
                  _bcast((H, H)), _bcast((1, H)), _bcast((H, 1)), _bcast((1, 1))],
        out_specs=[_rows(BE, H), _rows(BE, 1)],
        out_shape=[_sds((E, H)), _sds((E, 1))],
    )(pm, prel, attr, w, m1r, m2w, m2b, r1m, r1c, r2w, r2b, c0w, c0b, c1w, c1b)


def _tc_rel_u2s(out, rg, attr, r1m, r1c, r2w, r2b, t0w, t0b, t1w, t1b,
                p0w, p0b, p1w, p1b):
    def body(o_ref, rg_ref, a_ref, r1m_ref, r1c_ref, r2w_ref, r2b_ref,
             t0w_ref, t0b_ref, t1w_ref, t1b_ref, p0w_ref, p0b_ref,
             p1w_ref, p1b_ref, tl_ref, pl_ref2):
        r1 = _relu(rg_ref[:, H:] + _dot(o_ref[...], r1m_ref[...])
                   + a_ref[...] * r1c_ref[...])
        ua = _relu(_dot(r1, r2w_ref[...]) + r2b_ref[...])
        t1 = _relu(_dot(ua, t0w_ref[...]) + t0b_ref[...])
        tl = _leaky(_dot(t1, t1w_ref[...]) + t1b_ref[...])
        p1 = _relu(_dot(ua, p0w_ref[...]) + p0b_ref[...])
        plg = _leaky(_dot(p1, p1w_ref[...]) + p1b_ref[...])
        tl_ref[...] = tl
        pl_ref2[...] = plg

    return pl.pallas_call(
        body, grid=(NBE,),
        in_specs=[_rows(BE, H), _rows(BE, 2 * H), _rows(BE, 1),
                  _bcast((H, H)), _bcast((1, H)), _bcast((H, H)), _bcast((1, H)),
                  _bcast((H, H)), _bcast((1, H)), _bcast((H, 1)), _bcast((1, 1)),
                  _bcast((H, H)), _bcast((1, H)), _bcast((H, 1)), _bcast((1, 1))],
        out_specs=[_rows(BE, 1), _rows(BE, 1)],
        out_shape=[_sds((E, 1)), _sds((E, 1))],
    )(out, rg, attr, r1m, r1c, r2w, r2b, t0w, t0b, t1w, t1b, p0w, p0b, p1w, p1b)


def _tc_upd(agg2, x, u1, u2, ub):
    def body(p_ref, x_ref, u1_ref, u2_ref, ub_ref, o_ref):
        aggr = p_ref[0] + p_ref[1]
        o_ref[...] = _relu(_dot(aggr, u1_ref[...]) + _dot(x_ref[...], u2_ref[...])
                           + ub_ref[...])

    return pl.pallas_call(
        body, grid=(N // BN,),
        in_specs=[pl.BlockSpec((2, BN, H), lambda i: (0, i, 0)), _rows(BN, H),
                  _bcast((H, H)), _bcast((H, H)), _bcast((1, H))],
        out_specs=_rows(BN, H),
        out_shape=_sds((N, H)),
    )(agg2, x, u1, u2, ub)


def _global_max(sid, msb_v, mx16_v, sh_mx, m):
    msb_v[pl.ds(0, 16)] = m
    pltpu.sync_copy(msb_v, sh_mx.at[pl.ds(sid * 16, 16)])
    plsc.subcore_barrier()
    pltpu.sync_copy(sh_mx, mx16_v)
    mm = mx16_v[pl.ds(0, 16)]
    for r in range(1, 16):
        mm = jnp.maximum(mm, mx16_v[pl.ds(r * 16, 16)])
    return jnp.max(mm)


def _softmax_tail(sid, lg_v, idx_v, acc_v, t16_v, ss_v, sh_tabs, sh_S,
                  out_hbm, M):
    base = sid * EC

    @pl.loop(0, NSEGP // 16)
    def _(i):
        acc_v[pl.ds(i * 16, 16)] = jnp.zeros((16,), f32)

    @pl.loop(0, GP)
    def _(i):
        sl = pl.ds(i * 16, 16)
        e = jnp.exp(lg_v[sl] - M)
        lg_v[sl] = e
        plsc.addupdate_scatter(acc_v, [idx_v[sl]], e)

    pltpu.sync_copy(acc_v, sh_tabs.at[pl.ds(sid * NSEGP, NSEGP)])
    plsc.subcore_barrier()
    for r in range(16):
        pltpu.sync_copy(sh_tabs.at[pl.ds(r * NSEGP + sid * COLS, COLS)],
                        t16_v.at[pl.ds(r * COLS, COLS)])

    @pl.loop(0, COLS // 16)
    def _(j):
        s = t16_v[pl.ds(j * 16, 16)]
        for r in range(1, 16):
            s = s + t16_v[pl.ds(r * COLS + j * 16, 16)]
        ss_v[pl.ds(j * 16, 16)] = s

    pltpu.sync_copy(ss_v, sh_S.at[pl.ds(sid * COLS, COLS)])
    plsc.subcore_barrier()
    pltpu.sync_copy(sh_S, acc_v)

    @pl.loop(0, GP)
    def _(i):
        sl = pl.ds(i * 16, 16)
        s = plsc.load_gather(acc_v, [idx_v[sl]])
        lg_v[sl] = lg_v[sl] / (s + 1e-16)

    pltpu.sync_copy(lg_v, out_hbm.at[pl.ds(base, EC)])


def _sc_seg_softmax(logits, idx):
    mesh = plsc.VectorSubcoreMesh(**_MESH1)

    @functools.partial(
        pl.kernel, out_type=_sds((E,)), mesh=mesh,
        compiler_params=_sc_params(),
        scratch_types=[
            pltpu.VMEM((EC,), f32), pltpu.VMEM((EC,), i32),
            pltpu.VMEM((NSEGP,), f32), pltpu.VMEM((16 * COLS,), f32),
            pltpu.VMEM((COLS,), f32), pltpu.VMEM((16,), f32),
            pltpu.VMEM((256,), f32),
            pltpu.VMEM_SHARED((16 * NSEGP,), f32), pltpu.VMEM_SHARED((NSEGP,), f32),
            pltpu.VMEM_SHARED((256,), f32),
        ])
    def k(lg_hbm, idx_hbm, out_hbm, lg_v, idx_v, acc_v, t16_v, ss_v,
          msb_v, mx16_v, sh_tabs, sh_S, sh_mx):
        sid = lax.axis_index("s")
        base = sid * EC
        pltpu.sync_copy(lg_hbm.at[pl.ds(base, EC)], lg_v)
        pltpu.sync_copy(idx_hbm.at[pl.ds(base, EC)], idx_v)
        m = lax.fori_loop(
            0, GP,
            lambda i, m: jnp.maximum(m, lg_v[pl.ds(i * 16, 16)]),
            jnp.full((16,), -1e30, f32))
        M = _global_max(sid, msb_v, mx16_v, sh_mx, m)
        _softmax_tail(sid, lg_v, idx_v, acc_v, t16_v, ss_v, sh_tabs, sh_S,
                      out_hbm, M)

    return k(logits, idx)


def _sc_s2u_att_softmax(qs, rs, dst, src):
    mesh = plsc.VectorSubcoreMesh(**_MESH1)

    @functools.partial(
        pl.kernel, out_type=_sds((E,)), mesh=mesh,
        compiler_params=_sc_params(),
        scratch_types=[
            pltpu.VMEM((EC,), f32), pltpu.VMEM((EC,), i32),
            pltpu.VMEM((EC,), i32), pltpu.VMEM((N,), f32), pltpu.VMEM((N,), f32),
            pltpu.VMEM((NSEGP,), f32), pltpu.VMEM((16 * COLS,), f32),
            pltpu.VMEM((COLS,), f32), pltpu.VMEM((16,), f32),
            pltpu.VMEM((256,), f32),
            pltpu.VMEM_SHARED((16 * NSEGP,), f32), pltpu.VMEM_SHARED((NSEGP,), f32),
            pltpu.VMEM_SHARED((256,), f32),
        ])
    def k(qs_hbm, rs_hbm, dst_hbm, src_hbm, out_hbm, lg_v, idx_v, dst_v,
          qs_v, rs_v, acc_v, t16_v, ss_v, msb_v, mx16_v, sh_tabs, sh_S, sh_mx):
        sid = lax.axis_index("s")
        base = sid * EC
        pltpu.sync_copy(qs_hbm, qs_v)
        pltpu.sync_copy(rs_hbm, rs_v)
        pltpu.sync_copy(dst_hbm.at[pl.ds(base, EC)], dst_v)
        pltpu.sync_copy(src_hbm.at[pl.ds(base, EC)], idx_v)
        msb_v[pl.ds(0, 16)] = jnp.full((16,), -1e30, f32)

        @pl.loop(0, GP)
        def _(i):
            sl = pl.ds(i * 16, 16)
            q = plsc.load_gather(qs_v, [dst_v[sl]])
            r = plsc.load_gather(rs_v, [idx_v[sl]])
            l = _leaky(q + r)
            lg_v[sl] = l
            msb_v[pl.ds(0, 16)] = jnp.maximum(msb_v[pl.ds(0, 16)], l)

        M = _global_max(sid, msb_v, mx16_v, sh_mx, msb_v[pl.ds(0, 16)])
        _softmax_tail(sid, lg_v, idx_v, acc_v, t16_v, ss_v, sh_tabs, sh_S,
                      out_hbm, M)

    return k(qs, rs, dst, src)


def _sc_gather(tab, idx):
    D = tab.shape[1]
    RPW = E // 32
    CH = 200
    mesh = plsc.VectorSubcoreMesh(**_MESH2)

    @functools.partial(
        pl.kernel, out_type=_sds((E, D)), mesh=mesh,
        scratch_types=[pltpu.VMEM((RPW,), i32), pltpu.VMEM((CH, D), f32),
                       pltpu.SemaphoreType.DMA])
    def k(tab_hbm, idx_hbm, out_hbm, idx_v, rows_v, sem):
        wid = lax.axis_index("s") * 2 + lax.axis_index("c")
        base = wid * RPW
        pltpu.sync_copy(idx_hbm.at[pl.ds(base, RPW)], idx_v)

        @pl.loop(0, RPW // CH)
        def _(j):
            pltpu.async_copy(tab_hbm.at[idx_v.at[pl.ds(j * CH, CH)]],
                             rows_v, sem).wait()
            pltpu.sync_copy(rows_v, out_hbm.at[pl.ds(base + j * CH, CH)])

    return k(tab, idx)


def _sc_scatter_add(vals, idx):
    SCH = 200
    RPW = E // 32
    CPW = RPW // SCH
    mesh = plsc.VectorSubcoreMesh(**_MESH2)

    @functools.partial(
        pl.kernel, out_type=_sds((2, NSEGP, H)), mesh=mesh,
        scratch_types=[pltpu.VMEM((SCH,), i32), pltpu.VMEM((SCH, H), f32),
                       pltpu.VMEM((8, H), f32),
                       pltpu.VMEM_SHARED((NSEGP, H), f32)])
    def k(vals_hbm, idx_hbm, out_hbm, idx_c, vals_v, zb_v, sh_tab):
        cid = lax.axis_index("c")
        sid = lax.axis_index("s")
        wid = cid * 16 + sid
        for r in range(8):
            for jj in range(H // 16):
                zb_v[r, pl.ds(jj * 16, 16)] = jnp.zeros((16,), f32)

        @pl.loop(0, (NSEGP // 16) // 8)
        def _(j):
            pltpu.sync_copy(zb_v, sh_tab.at[pl.ds(sid * (NSEGP // 16) + j * 8, 8)])

        plsc.subcore_barrier()
        ebase = wid * RPW

        @pl.loop(0, CPW)
        def _(j):
            pltpu.sync_copy(idx_hbm.at[pl.ds(ebase + j * SCH, SCH)], idx_c)
            pltpu.sync_copy(vals_hbm.at[pl.ds(ebase + j * SCH, SCH)], vals_v)
            pltpu.sync_copy(vals_v, sh_tab.at[idx_c], add=True)

        plsc.subcore_barrier()
        pltpu.sync_copy(sh_tab.at[pl.ds(sid * (NSEGP // 16), NSEGP // 16)],
                        out_hbm.at[cid].at[pl.ds(sid * (NSEGP // 16), NSEGP // 16)])

    return k(vals, idx)


def kernel(x_user, x_server, edge_index_u2s, edge_index_s2u,
           edge_attr_u2s, edge_attr_s2u, params):
    us_src, us_dst = edge_index_u2s[0], edge_index_u2s[1]
    su_src, su_dst = edge_index_s2u[0], edge_index_s2u[1]
    ea_us = edge_attr_u2s
    ea_su = edge_attr_s2u

    def row(b):
        return b.reshape(1, H)

    p = params
    xu = _tc_mlp2(x_user, p["user_enc"][0]["w"], row(p["user_enc"][0]["b"]),
                  p["user_enc"][1]["w"], row(p["user_enc"][1]["b"]))
    xs = _tc_mlp2(x_server, p["server_enc"][0]["w"], row(p["server_enc"][0]["b"]),
                  p["server_enc"][1]["w"], row(p["server_enc"][1]["b"]))

    tlogit = plogit = clogit = None

    for li, lp in enumerate(p["layers"]):
        last = li == len(p["layers"]) - 1
        ps, pu = lp["s2u"], lp["u2s"]

        wa, ba = ps["att"][0]["w"], ps["att"][0]["b"]
        qs_w = ps["Wq"]["w"] @ wa[:H]
        qs_b = (ps["Wq"]["b"] @ wa[:H] + ba).reshape(1, 1)
        rs_w = ps["Wr"]["w"] @ wa[H:]
        rs_b = (ps["Wr"]["b"] @ wa[H:]).reshape(1, 1)
        A0w, A0b = pu["att"][0]["w"], pu["att"][0]["b"]
        Qs_w = pu["Wq"]["w"] @ A0w[:H]
        Qs_b = (pu["Wq"]["b"] @ A0w[:H] + A0b).reshape(1, H)
        Ru_w = pu["Wr"]["w"] @ A0w[H:]
        Ru_b = (pu["Wr"]["b"] @ A0w[H:]).reshape(1, H)
        M1w, M1b = ps["msg"][0]["w"], ps["msg"][0]["b"]
        M0w, M0b = pu["msg"][0]["w"], pu["msg"][0]["b"]
        if last:
            R1s, R1sb = ps["rel"][0]["w"], ps["rel"][0]["b"]
            R1u, R1ub = pu["rel"][0]["w"], pu["rel"][0]["b"]
            wu_cat = jnp.concatenate([Ru_w, R1u[:H], M1w[:H], qs_w], axis=1)
            bu_cat = jnp.concatenate(
                [Ru_b, R1ub.reshape(1, H), M1b.reshape(1, H), qs_b], axis=1)
            ws_cat = jnp.concatenate([Qs_w, M0w[:H], R1s[:H], rs_w], axis=1)
            bs_cat = jnp.concatenate(
                [Qs_b, M0b.reshape(1, H), R1sb.reshape(1, H), rs_b], axis=1)
            tab_us_src, pmsg_u, qs_u = _tc_node_pre(
                xu, wu_cat, bu_cat, [2 * H, H, 1])
            tab_us_dst, tab_su_src, rs_s = _tc_node_pre(
                xs, ws_cat, bs_cat, [2 * H, H, 1])
        else:
            wu_cat = jnp.concatenate([Ru_w, M1w[:H], qs_w], axis=1)
            bu_cat = jnp.concatenate(
                [Ru_b, M1b.reshape(1, H), qs_b], axis=1)
            ws_cat = jnp.concatenate([Qs_w, M0w[:H], rs_w], axis=1)
            bs_cat = jnp.concatenate([Qs_b, M0b.reshape(1, H), rs_b], axis=1)
            tab_us_src, pmsg_u, qs_u = _tc_node_pre(
                xu, wu_cat, bu_cat, [H, H, 1])
            tab_us_dst, rs_s = _tc_node_pre(xs, ws_cat, bs_cat, [2 * H, 1])
            tab_su_src = None

        pmg_u = _sc_gather(pmsg_u, su_dst)
        qpg = _sc_gather(tab_us_dst, us_dst)
        rg = _sc_gather(tab_us_src, us_src)
        prelg_s = _sc_gather(tab_su_src, su_src) if last else None

        w_su = _sc_s2u_att_softmax(qs_u.reshape(N,), rs_s.reshape(N,),
                                   su_dst, su_src)
        lg_us = _tc_att_u2s(qpg, rg, pu["att"][1]["w"],
                            pu["att"][1]["b"].reshape(1, 1))
        w_us = _sc_seg_softmax(lg_us.reshape(E,), us_src)

        if last:
            out_su, clogit = _tc_msg_s2u_l1(
                pmg_u, prelg_s, ea_su, w_su.reshape(E, 1),
                row(M1w[H]), ps["msg"][1]["w"], row(ps["msg"][1]["b"]),
                R1s[H:2 * H], row(R1s[2 * H]),
                ps["rel"][1]["w"], row(ps["rel"][1]["b"]),
                p["comp_mlp"][0]["w"], row(p["comp_mlp"][0]["b"]),
                p["comp_mlp"][1]["w"], p["comp_mlp"][1]["b"].reshape(1, 1))
        else:
            out_su = _tc_msg_s2u_l0(
                pmg_u, ea_su, w_su.reshape(E, 1), row(M1w[H]),
                ps["msg"][1]["w"], row(ps["msg"][1]["b"]))
        out_us = _tc_out_u2s(qpg, ea_us, w_us.reshape(E, 1), row(M0w[H]))
        if last:
            tlogit, plogit = _tc_rel_u2s(
                out_us, rg, ea_us, R1u[H:2 * H], row(R1u[2 * H]),
                pu["rel"][1]["w"], row(pu["rel"][1]["b"]),
                p["task_mlp"][0]["w"], row(p["task_mlp"][0]["b"]),
                p["task_mlp"][1]["w"], p["task_mlp"][1]["b"].reshape(1, 1),
                p["power_mlp"][0]["w"], row(p["power_mlp"][0]["b"]),
                p["power_mlp"][1]["w"], p["power_mlp"][1]["b"].reshape(1, 1))

        agg_u = _sc_scatter_add(out_su, su_dst)
        agg_s = _sc_scatter_add(out_us, us_dst)
        Uw, Ub = ps["upd"]["w"], ps["upd"]["b"]
        xu = _tc_upd(agg_u, xu, Uw[:H], Uw[H:], row(Ub))
        Uw, Ub = pu["upd"]["w"], pu["upd"]["b"]
        xs = _tc_upd(agg_s, xs, Uw[:H], Uw[H:], row(Ub))

    task = _sc_seg_softmax(tlogit.reshape(E,), us_src)
    power = _sc_seg_softmax(plogit.reshape(E,), us_src)
    comp = _sc_seg_softmax(clogit.reshape(E,), su_src)
    return task.reshape(E, 1), power.reshape(E, 1), comp.reshape(E, 1)


# speedup vs baseline: 8.6764x; 1.0163x over previous
import dataclasses
import functools

import jax
import jax.numpy as jnp
from jax import lax
from jax.experimental import pallas as pl
from jax.experimental.pallas import tpu as pltpu
from jax.experimental.pallas import tpu_sc as plsc

f32 = jnp.float32
i32 = jnp.int32

H = 128
ALPHA = 0.2
N = 10000
E = 160000
BN = 2000
BE = 2000
NBE = E // BE
NSEGP = 10240
EC = E // 16
GP = EC // 16
COLS = NSEGP // 16

_MESH1 = dict(core_axis_name="c", subcore_axis_name="s", num_cores=1)
_MESH2 = dict(core_axis_name="c", subcore_axis_name="s")


def _sc_params():
    cp = pltpu.CompilerParams()
    if "needs_layout_passes" in pltpu.CompilerParams.__dataclass_fields__:
        cp = dataclasses.replace(cp, needs_layout_passes=False)
    return cp


def _relu(x):
    return jnp.maximum(x, 0.0)


def _leaky(x):
    return jnp.maximum(x, ALPHA * x)


def _dot(a, b):
    return jnp.dot(a, b, preferred_element_type=f32)


def _rows(b, d):
    return pl.BlockSpec((b, d), lambda i: (i, 0))


def _bcast(shape):
    return pl.BlockSpec(shape, lambda i: tuple(0 for _ in shape))


def _sds(shape):
    return jax.ShapeDtypeStruct(shape, f32)


def _tc_mlp2(x, w1, b1, w2, b2):
    def body(x_ref, w1_ref, b1_ref, w2_ref, b2_ref, o_ref):
        h = _relu(_dot(x_ref[...], w1_ref[...]) + b1_ref[...])
        o_ref[...] = _relu(_dot(h, w2_ref[...]) + b2_ref[...])

    return pl.pallas_call(
        body, grid=(N // BN,),
        in_specs=[_rows(BN, H), _bcast((H, H)), _bcast((1, H)),
                  _bcast((H, H)), _bcast((1, H))],
        out_specs=_rows(BN, H),
        out_shape=_sds((N, H)),
    )(x, w1, b1, w2, b2)


def _tc_node_pre(x, w, b, out_dims):
    D = sum(out_dims)

    def body(x_ref, w_ref, b_ref, *outs):
        y = _dot(x_ref[...], w_ref[...]) + b_ref[...]
        off = 0
        for o_ref, d in zip(outs, out_dims):
            o_ref[...] = y[:, off:off + d]
            off += d

    return pl.pallas_call(
        body, grid=(N // BN,),
        in_specs=[_rows(BN, H), _bcast((H, D)), _bcast((1, D))],
        out_specs=[_rows(BN, d) for d in out_dims],
        out_shape=[_sds((N, d)) for d in out_dims],
    )(x, w, b)


def _tc_att_u2s(qp, rg, a1w, a1b):
    def body(qp_ref, rg_ref, a1w_ref, a1b_ref, lg_ref):
        ah = _relu(qp_ref[:, :H] + rg_ref[:, :H])
        lg_ref[...] = _leaky(_dot(ah, a1w_ref[...]) + a1b_ref[...])

    return pl.pallas_call(
        body, grid=(NBE,),
        in_specs=[_rows(BE, qp.shape[1]), _rows(BE, rg.shape[1]),
                  _bcast((H, 1)), _bcast((1, 1))],
        out_specs=_rows(BE, 1),
        out_shape=_sds((E, 1)),
    )(qp, rg, a1w, a1b)


def _tc_out_u2s(qp, attr, w, m0r):
    def body(qp_ref, a_ref, w_ref, m_ref, o_ref):
        o_ref[...] = w_ref[...] * _relu(qp_ref[:, H:] + a_ref[...] * m_ref[...])

    return pl.pallas_call(
        body, grid=(NBE,),
        in_specs=[_rows(BE, 2 * H), _rows(BE, 1), _rows(BE, 1), _bcast((1, H))],
        out_specs=_rows(BE, H),
        out_shape=_sds((E, H)),
    )(qp, attr, w, m0r)


def _tc_msg_s2u_l0(pm, attr, w, m1r, m2w, m2b):
    def body(pm_ref, a_ref, w_ref, m1_ref, m2w_ref, m2b_ref, o_ref):
        h1 = _relu(pm_ref[...] + a_ref[...] * m1_ref[...])
        o_ref[...] = w_ref[...] * _relu(_dot(h1, m2w_ref[...]) + m2b_ref[...])

    return pl.pallas_call(
        body, grid=(NBE,),
        in_specs=[_rows(BE, H), _rows(BE, 1), _rows(BE, 1), _bcast((1, H)),
                  _bcast((H, H)), _bcast((1, H))],
        out_specs=_rows(BE, H),
        out_shape=_sds((E, H)),
    )(pm, attr, w, m1r, m2w, m2b)


def _tc_msg_s2u_l1(pm, prel, attr, w, m1r, m2w, m2b, r1m, r1c, r2w, r2b,
                   c0w, c0b, c1w, c1b):
    def body(pm_ref, pr_ref, a_ref, w_ref, m1_ref, m2w_ref, m2b_ref,
             r1m_ref, r1c_ref, r2w_ref, r2b_ref, c0w_ref, c0b_ref,
             c1w_ref, c1b_ref, o_ref, cl_ref):
        h1 = _relu(pm_ref[...] + a_ref[...] * m1_ref[...])
        out = w_ref[...] * _relu(_dot(h1, m2w_ref[...]) + m2b_ref[...])
        o_ref[...] = out
        r1 = _relu(pr_ref[...] + _dot(out, r1m_ref[...])
                   + a_ref[...] * r1c_ref[...])
        sattr = _relu(_dot(r1, r2w_ref[...]) + r2b_ref[...])
        c1 = _relu(_dot(sattr, c0w_ref[...]) + c0b_ref[...])
        cl_ref[...] = _leaky(_dot(c1, c1w_ref[...]) + c1b_ref[...])

    return pl.pallas_call(
        body, grid=(NBE,),
        in_specs=[_rows(BE, H), _rows(BE, H), _rows(BE, 1), _rows(BE, 1),
                  _bcast((1, H)), _bcast((H, H)), _bcast((1, H)),
                  _bcast((H, H)), _bcast((1, H)), _bcast((H, H)), _bcast((1, H)),
                  _bcast((H, H)), _bcast((1, H)), _bcast((H, 1)), _bcast((1, 1))],
        out_specs=[_rows(BE, H), _rows(BE, 1)],
        out_shape=[_sds((E, H)), _sds((E, 1))],
    )(pm, prel, attr, w, m1r, m2w, m2b, r1m, r1c, r2w, r2b, c0w, c0b, c1w, c1b)


def _tc_rel_u2s(out, rg, attr, r1m, r1c, r2w, r2b, t0w, t0b, t1w, t1b,
                p0w, p0b, p1w, p1b):
    def body(o_ref, rg_ref, a_ref, r1m_ref, r1c_ref, r2w_ref, r2b_ref,
             t0w_ref, t0b_ref, t1w_ref, t1b_ref, p0w_ref, p0b_ref,
             p1w_ref, p1b_ref, tl_ref, pl_ref2):
        r1 = _relu(rg_ref[:, H:] + _dot(o_ref[...], r1m_ref[...])
                   + a_ref[...] * r1c_ref[...])
        ua = _relu(_dot(r1, r2w_ref[...]) + r2b_ref[...])
        t1 = _relu(_dot(ua, t0w_ref[...]) + t0b_ref[...])
        tl = _leaky(_dot(t1, t1w_ref[...]) + t1b_ref[...])
        p1 = _relu(_dot(ua, p0w_ref[...]) + p0b_ref[...])
        plg = _leaky(_dot(p1, p1w_ref[...]) + p1b_ref[...])
        tl_ref[...] = tl
        pl_ref2[...] = plg

    return pl.pallas_call(
        body, grid=(NBE,),
        in_specs=[_rows(BE, H), _rows(BE, 2 * H), _rows(BE, 1),
                  _bcast((H, H)), _bcast((1, H)), _bcast((H, H)), _bcast((1, H)),
                  _bcast((H, H)), _bcast((1, H)), _bcast((H, 1)), _bcast((1, 1)),
                  _bcast((H, H)), _bcast((1, H)), _bcast((H, 1)), _bcast((1, 1))],
        out_specs=[_rows(BE, 1), _rows(BE, 1)],
        out_shape=[_sds((E, 1)), _sds((E, 1))],
    )(out, rg, attr, r1m, r1c, r2w, r2b, t0w, t0b, t1w, t1b, p0w, p0b, p1w, p1b)


def _tc_upd(agg2, x, u1, u2, ub):
    def body(p_ref, x_ref, u1_ref, u2_ref, ub_ref, o_ref):
        aggr = p_ref[0] + p_ref[1]
        o_ref[...] = _relu(_dot(aggr, u1_ref[...]) + _dot(x_ref[...], u2_ref[...])
                           + ub_ref[...])

    return pl.pallas_call(
        body, grid=(N // BN,),
        in_specs=[pl.BlockSpec((2, BN, H), lambda i: (0, i, 0)), _rows(BN, H),
                  _bcast((H, H)), _bcast((H, H)), _bcast((1, H))],
        out_specs=_rows(BN, H),
        out_shape=_sds((N, H)),
    )(agg2, x, u1, u2, ub)


def _global_max(sid, msb_v, mx16_v, sh_mx, m):
    msb_v[pl.ds(0, 16)] = m
    pltpu.sync_copy(msb_v, sh_mx.at[pl.ds(sid * 16, 16)])
    plsc.subcore_barrier()
    pltpu.sync_copy(sh_mx, mx16_v)
    mm = mx16_v[pl.ds(0, 16)]
    for r in range(1, 16):
        mm = jnp.maximum(mm, mx16_v[pl.ds(r * 16, 16)])
    return jnp.max(mm)


def _softmax_tail(sid, lg_v, idx_v, acc_v, t16_v, ss_v, sh_tabs, sh_S,
                  out_hbm, M):
    base = sid * EC

    @pl.loop(0, NSEGP // 16)
    def _(i):
        acc_v[pl.ds(i * 16, 16)] = jnp.zeros((16,), f32)

    @pl.loop(0, GP)
    def _(i):
        sl = pl.ds(i * 16, 16)
        e = jnp.exp(lg_v[sl] - M)
        lg_v[sl] = e
        plsc.addupdate_scatter(acc_v, [idx_v[sl]], e)

    pltpu.sync_copy(acc_v, sh_tabs.at[pl.ds(sid * NSEGP, NSEGP)])
    plsc.subcore_barrier()
    for r in range(16):
        pltpu.sync_copy(sh_tabs.at[pl.ds(r * NSEGP + sid * COLS, COLS)],
                        t16_v.at[pl.ds(r * COLS, COLS)])

    @pl.loop(0, COLS // 16)
    def _(j):
        s = t16_v[pl.ds(j * 16, 16)]
        for r in range(1, 16):
            s = s + t16_v[pl.ds(r * COLS + j * 16, 16)]
        ss_v[pl.ds(j * 16, 16)] = s

    pltpu.sync_copy(ss_v, sh_S.at[pl.ds(sid * COLS, COLS)])
    plsc.subcore_barrier()
    pltpu.sync_copy(sh_S, acc_v)

    @pl.loop(0, GP)
    def _(i):
        sl = pl.ds(i * 16, 16)
        s = plsc.load_gather(acc_v, [idx_v[sl]])
        lg_v[sl] = lg_v[sl] / (s + 1e-16)

    pltpu.sync_copy(lg_v, out_hbm.at[pl.ds(base, EC)])


def _sc_seg_softmax(logits, idx):
    mesh = plsc.VectorSubcoreMesh(**_MESH1)

    @functools.partial(
        pl.kernel, out_type=_sds((E,)), mesh=mesh,
        compiler_params=_sc_params(),
        scratch_types=[
            pltpu.VMEM((EC,), f32), pltpu.VMEM((EC,), i32),
            pltpu.VMEM((NSEGP,), f32), pltpu.VMEM((16 * COLS,), f32),
            pltpu.VMEM((COLS,), f32), pltpu.VMEM((16,), f32),
            pltpu.VMEM((256,), f32),
            pltpu.VMEM_SHARED((16 * NSEGP,), f32), pltpu.VMEM_SHARED((NSEGP,), f32),
            pltpu.VMEM_SHARED((256,), f32),
        ])
    def k(lg_hbm, idx_hbm, out_hbm, lg_v, idx_v, acc_v, t16_v, ss_v,
          msb_v, mx16_v, sh_tabs, sh_S, sh_mx):
        sid = lax.axis_index("s")
        base = sid * EC
        pltpu.sync_copy(lg_hbm.at[pl.ds(base, EC)], lg_v)
        pltpu.sync_copy(idx_hbm.at[pl.ds(base, EC)], idx_v)
        m = lax.fori_loop(
            0, GP,
            lambda i, m: jnp.maximum(m, lg_v[pl.ds(i * 16, 16)]),
            jnp.full((16,), -1e30, f32))
        M = _global_max(sid, msb_v, mx16_v, sh_mx, m)
        _softmax_tail(sid, lg_v, idx_v, acc_v, t16_v, ss_v, sh_tabs, sh_S,
                      out_hbm, M)

    return k(logits, idx)


def _sc_s2u_att_softmax(qs, rs, dst, src):
    mesh = plsc.VectorSubcoreMesh(**_MESH1)

    @functools.partial(
        pl.kernel, out_type=_sds((E,)), mesh=mesh,
        compiler_params=_sc_params(),
        scratch_types=[
            pltpu.VMEM((EC,), f32), pltpu.VMEM((EC,), i32),
            pltpu.VMEM((EC,), i32), pltpu.VMEM((N,), f32), pltpu.VMEM((N,), f32),
            pltpu.VMEM((NSEGP,), f32), pltpu.VMEM((16 * COLS,), f32),
            pltpu.VMEM((COLS,), f32), pltpu.VMEM((16,), f32),
            pltpu.VMEM((256,), f32),
            pltpu.VMEM_SHARED((16 * NSEGP,), f32), pltpu.VMEM_SHARED((NSEGP,), f32),
            pltpu.VMEM_SHARED((256,), f32),
        ])
    def k(qs_hbm, rs_hbm, dst_hbm, src_hbm, out_hbm, lg_v, idx_v, dst_v,
          qs_v, rs_v, acc_v, t16_v, ss_v, msb_v, mx16_v, sh_tabs, sh_S, sh_mx):
        sid = lax.axis_index("s")
        base = sid * EC
        pltpu.sync_copy(qs_hbm, qs_v)
        pltpu.sync_copy(rs_hbm, rs_v)
        pltpu.sync_copy(dst_hbm.at[pl.ds(base, EC)], dst_v)
        pltpu.sync_copy(src_hbm.at[pl.ds(base, EC)], idx_v)
        msb_v[pl.ds(0, 16)] = jnp.full((16,), -1e30, f32)

        @pl.loop(0, GP)
        def _(i):
            sl = pl.ds(i * 16, 16)
            q = plsc.load_gather(qs_v, [dst_v[sl]])
            r = plsc.load_gather(rs_v, [idx_v[sl]])
            l = _leaky(q + r)
            lg_v[sl] = l
            msb_v[pl.ds(0, 16)] = jnp.maximum(msb_v[pl.ds(0, 16)], l)

        M = _global_max(sid, msb_v, mx16_v, sh_mx, msb_v[pl.ds(0, 16)])
        _softmax_tail(sid, lg_v, idx_v, acc_v, t16_v, ss_v, sh_tabs, sh_S,
                      out_hbm, M)

    return k(qs, rs, dst, src)


def _sc_gather(tab, idx):
    D = tab.shape[1]
    RPW = E // 32
    CH = 200
    NCH = RPW // CH
    mesh = plsc.VectorSubcoreMesh(**_MESH2)

    @functools.partial(
        pl.kernel, out_type=_sds((E, D)), mesh=mesh,
        scratch_types=[pltpu.VMEM((RPW,), i32),
                       pltpu.VMEM((CH, D), f32), pltpu.VMEM((CH, D), f32),
                       pltpu.SemaphoreType.DMA, pltpu.SemaphoreType.DMA])
    def k(tab_hbm, idx_hbm, out_hbm, idx_v, rows0, rows1, sem0, sem1):
        wid = lax.axis_index("s") * 2 + lax.axis_index("c")
        base = wid * RPW
        pltpu.sync_copy(idx_hbm.at[pl.ds(base, RPW)], idx_v)

        def start(j, buf, sem):
            return pltpu.async_copy(
                tab_hbm.at[idx_v.at[pl.ds(j * CH, CH)]], buf, sem)

        def flush(j, buf):
            pltpu.sync_copy(buf, out_hbm.at[pl.ds(base + j * CH, CH)])

        start(0, rows0, sem0)

        @pl.loop(0, NCH // 2)
        def _(kk):
            j = kk * 2
            pltpu.make_async_copy(
                tab_hbm.at[idx_v.at[pl.ds(j * CH, CH)]], rows0, sem0).wait()
            start(j + 1, rows1, sem1)
            flush(j, rows0)
            pltpu.make_async_copy(
                tab_hbm.at[idx_v.at[pl.ds((j + 1) * CH, CH)]], rows1, sem1).wait()

            @pl.when(j + 2 < NCH)
            def _():
                start(j + 2, rows0, sem0)

            flush(j + 1, rows1)

        pltpu.make_async_copy(
            tab_hbm.at[idx_v.at[pl.ds((NCH - 1) * CH, CH)]], rows0, sem0).wait()
        flush(NCH - 1, rows0)

    return k(tab, idx)


def _sc_scatter_add(vals, idx):
    SCH = 200
    RPW = E // 32
    CPW = RPW // SCH
    mesh = plsc.VectorSubcoreMesh(**_MESH2)

    @functools.partial(
        pl.kernel, out_type=_sds((2, NSEGP, H)), mesh=mesh,
        scratch_types=[pltpu.VMEM((SCH,), i32), pltpu.VMEM((SCH, H), f32),
                       pltpu.VMEM((8, H), f32),
                       pltpu.VMEM_SHARED((NSEGP, H), f32)])
    def k(vals_hbm, idx_hbm, out_hbm, idx_c, vals_v, zb_v, sh_tab):
        cid = lax.axis_index("c")
        sid = lax.axis_index("s")
        wid = cid * 16 + sid
        for r in range(8):
            for jj in range(H // 16):
                zb_v[r, pl.ds(jj * 16, 16)] = jnp.zeros((16,), f32)

        @pl.loop(0, (NSEGP // 16) // 8)
        def _(j):
            pltpu.sync_copy(zb_v, sh_tab.at[pl.ds(sid * (NSEGP // 16) + j * 8, 8)])

        plsc.subcore_barrier()
        ebase = wid * RPW

        @pl.loop(0, CPW)
        def _(j):
            pltpu.sync_copy(idx_hbm.at[pl.ds(ebase + j * SCH, SCH)], idx_c)
            pltpu.sync_copy(vals_hbm.at[pl.ds(ebase + j * SCH, SCH)], vals_v)
            pltpu.sync_copy(vals_v, sh_tab.at[idx_c], add=True)

        plsc.subcore_barrier()
        pltpu.sync_copy(sh_tab.at[pl.ds(sid * (NSEGP // 16), NSEGP // 16)],
                        out_hbm.at[cid].at[pl.ds(sid * (NSEGP // 16), NSEGP // 16)])

    return k(vals, idx)


def kernel(x_user, x_server, edge_index_u2s, edge_index_s2u,
           edge_attr_u2s, edge_attr_s2u, params):
    us_src, us_dst = edge_index_u2s[0], edge_index_u2s[1]
    su_src, su_dst = edge_index_s2u[0], edge_index_s2u[1]
    ea_us = edge_attr_u2s
    ea_su = edge_attr_s2u

    def row(b):
        return b.reshape(1, H)

    p = params
    xu = _tc_mlp2(x_user, p["user_enc"][0]["w"], row(p["user_enc"][0]["b"]),
                  p["user_enc"][1]["w"], row(p["user_enc"][1]["b"]))
    xs = _tc_mlp2(x_server, p["server_enc"][0]["w"], row(p["server_enc"][0]["b"]),
                  p["server_enc"][1]["w"], row(p["server_enc"][1]["b"]))

    tlogit = plogit = clogit = None

    for li, lp in enumerate(p["layers"]):
        last = li == len(p["layers"]) - 1
        ps, pu = lp["s2u"], lp["u2s"]

        wa, ba = ps["att"][0]["w"], ps["att"][0]["b"]
        qs_w = ps["Wq"]["w"] @ wa[:H]
        qs_b = (ps["Wq"]["b"] @ wa[:H] + ba).reshape(1, 1)
        rs_w = ps["Wr"]["w"] @ wa[H:]
        rs_b = (ps["Wr"]["b"] @ wa[H:]).reshape(1, 1)
        A0w, A0b = pu["att"][0]["w"], pu["att"][0]["b"]
        Qs_w = pu["Wq"]["w"] @ A0w[:H]
        Qs_b = (pu["Wq"]["b"] @ A0w[:H] + A0b).reshape(1, H)
        Ru_w = pu["Wr"]["w"] @ A0w[H:]
        Ru_b = (pu["Wr"]["b"] @ A0w[H:]).reshape(1, H)
        M1w, M1b = ps["msg"][0]["w"], ps["msg"][0]["b"]
        M0w, M0b = pu["msg"][0]["w"], pu["msg"][0]["b"]
        if last:
            R1s, R1sb = ps["rel"][0]["w"], ps["rel"][0]["b"]
            R1u, R1ub = pu["rel"][0]["w"], pu["rel"][0]["b"]
            wu_cat = jnp.concatenate([Ru_w, R1u[:H], M1w[:H], qs_w], axis=1)
            bu_cat = jnp.concatenate(
                [Ru_b, R1ub.reshape(1, H), M1b.reshape(1, H), qs_b], axis=1)
            ws_cat = jnp.concatenate([Qs_w, M0w[:H], R1s[:H], rs_w], axis=1)
            bs_cat = jnp.concatenate(
                [Qs_b, M0b.reshape(1, H), R1sb.reshape(1, H), rs_b], axis=1)
            tab_us_src, pmsg_u, qs_u = _tc_node_pre(
                xu, wu_cat, bu_cat, [2 * H, H, 1])
            tab_us_dst, tab_su_src, rs_s = _tc_node_pre(
                xs, ws_cat, bs_cat, [2 * H, H, 1])
        else:
            wu_cat = jnp.concatenate([Ru_w, M1w[:H], qs_w], axis=1)
            bu_cat = jnp.concatenate(
                [Ru_b, M1b.reshape(1, H), qs_b], axis=1)
            ws_cat = jnp.concatenate([Qs_w, M0w[:H], rs_w], axis=1)
            bs_cat = jnp.concatenate([Qs_b, M0b.reshape(1, H), rs_b], axis=1)
            tab_us_src, pmsg_u, qs_u = _tc_node_pre(
                xu, wu_cat, bu_cat, [H, H, 1])
            tab_us_dst, rs_s = _tc_node_pre(xs, ws_cat, bs_cat, [2 * H, 1])
            tab_su_src = None

        pmg_u = _sc_gather(pmsg_u, su_dst)
        qpg = _sc_gather(tab_us_dst, us_dst)
        rg = _sc_gather(tab_us_src, us_src)
        prelg_s = _sc_gather(tab_su_src, su_src) if last else None

        w_su = _sc_s2u_att_softmax(qs_u.reshape(N,), rs_s.reshape(N,),
                                   su_dst, su_src)
        lg_us = _tc_att_u2s(qpg, rg, pu["att"][1]["w"],
                            pu["att"][1]["b"].reshape(1, 1))
        w_us = _sc_seg_softmax(lg_us.reshape(E,), us_src)

        if last:
            out_su, clogit = _tc_msg_s2u_l1(
                pmg_u, prelg_s, ea_su, w_su.reshape(E, 1),
                row(M1w[H]), ps["msg"][1]["w"], row(ps["msg"][1]["b"]),
                R1s[H:2 * H], row(R1s[2 * H]),
                ps["rel"][1]["w"], row(ps["rel"][1]["b"]),
                p["comp_mlp"][0]["w"], row(p["comp_mlp"][0]["b"]),
                p["comp_mlp"][1]["w"], p["comp_mlp"][1]["b"].reshape(1, 1))
        else:
            out_su = _tc_msg_s2u_l0(
                pmg_u, ea_su, w_su.reshape(E, 1), row(M1w[H]),
                ps["msg"][1]["w"], row(ps["msg"][1]["b"]))
        out_us = _tc_out_u2s(qpg, ea_us, w_us.reshape(E, 1), row(M0w[H]))
        if last:
            tlogit, plogit = _tc_rel_u2s(
                out_us, rg, ea_us, R1u[H:2 * H], row(R1u[2 * H]),
                pu["rel"][1]["w"], row(pu["rel"][1]["b"]),
                p["task_mlp"][0]["w"], row(p["task_mlp"][0]["b"]),
                p["task_mlp"][1]["w"], p["task_mlp"][1]["b"].reshape(1, 1),
                p["power_mlp"][0]["w"], row(p["power_mlp"][0]["b"]),
                p["power_mlp"][1]["w"], p["power_mlp"][1]["b"].reshape(1, 1))

        agg_u = _sc_scatter_add(out_su, su_dst)
        agg_s = _sc_scatter_add(out_us, us_dst)
        Uw, Ub = ps["upd"]["w"], ps["upd"]["b"]
        xu = _tc_upd(agg_u, xu, Uw[:H], Uw[H:], row(Ub))
        Uw, Ub = pu["upd"]["w"], pu["upd"]["b"]
        xs = _tc_upd(agg_s, xs, Uw[:H], Uw[H:], row(Ub))

    task = _sc_seg_softmax(tlogit.reshape(E,), us_src)
    power = _sc_seg_softmax(plogit.reshape(E,), us_src)
    comp = _sc_seg_softmax(clogit.reshape(E,), su_src)
    return task.reshape(E, 1), power.reshape(E, 1), comp.reshape(E, 1)
